# Initial kernel scaffold; baseline (speedup 1.0000x reference)
import jax
import jax.numpy as jnp
from jax.experimental import pallas as pl


def kernel(input_, offsets, emb_weights, emb_table):
    raise NotImplementedError("write your pallas kernel here")


# speedup vs baseline: 89.9386x; 89.9386x over previous
import functools

import jax
import jax.numpy as jnp
from jax import lax
from jax.experimental import pallas as pl
from jax.experimental.pallas import tpu as pltpu
from jax.experimental.pallas import tpu_sc as plsc

N_TOKENS = 204800
N_BAGS = 4096
VOCAB = 100000
EMB_DIM = 64

NC = 2
NS = 16
NW = NC * NS
NBW = N_BAGS // NW
CHUNK = 256
NIDX = 128
LANES = 16
PAIR = 2 * EMB_DIM


def _splat(val):
    return jnp.full((LANES,), val, jnp.int32)


def _body(inp_hbm, offs_hbm, w_hbm, tab_hbm, out_hbm,
          offs_v, offs2_v, idx_v, pair_v, poff_v, w_v, rows_v, acc_v, sem):
    cid = lax.axis_index("c")
    sid = lax.axis_index("s")
    wid = sid * NC + cid
    b0 = wid * NBW

    pltpu.sync_copy(offs_hbm.at[pl.ds(b0, NBW)], offs_v)
    nxt = jnp.minimum(b0 + NBW, N_BAGS - LANES)
    pltpu.sync_copy(offs_hbm.at[pl.ds(nxt, LANES)], offs2_v)

    t0 = offs_v[pl.ds(0, LANES)][0]
    t1 = jnp.where(wid == NW - 1, N_TOKENS, offs2_v[pl.ds(0, LANES)][0])

    zero16f = jnp.zeros((LANES,), jnp.float32)

    def zbody(i, _):
        for k in range(EMB_DIM // LANES):
            acc_v[i, pl.ds(k * LANES, LANES)] = zero16f
        return 0

    lax.fori_loop(0, NBW, zbody, 0)

    c_start = t0 // CHUNK
    c_end = (t1 + CHUNK - 1) // CHUNK

    def chunk_body(c, carry):
        p, cur, a0, a1, a2, a3 = carry
        s = c * CHUNK
        pltpu.sync_copy(inp_hbm.at[pl.ds(s, CHUNK)], idx_v)
        pltpu.sync_copy(w_hbm.at[pl.ds(s, CHUNK)], w_v)

        def pbody(g, _):
            v = idx_v[pl.ds(g * LANES, LANES)]
            pair_v[pl.ds(g * LANES, LANES)] = v >> 1
            poff_v[pl.ds(g * LANES, LANES)] = (v & 1) * EMB_DIM
            return 0

        lax.fori_loop(0, CHUNK // LANES, pbody, 0)

        cps = [
            pltpu.async_copy(
                tab_hbm.at[pair_v.at[pl.ds(j * NIDX, NIDX)]],
                rows_v.at[pl.ds(j * NIDX, NIDX)], sem)
            for j in range(CHUNK // NIDX)
        ]
        for cp in cps:
            cp.wait()

        hi = jnp.minimum(t1, s + CHUNK)

        S = jnp.int32(0)
        for step in (64, 32, 16, 8, 4, 2, 1, 1):
            idx = S + step
            probe = jnp.minimum(idx - 1, NBW - 1)
            val = plsc.load_gather(offs_v, [_splat(probe)])[0]
            S = jnp.where(jnp.logical_and(idx <= NBW, val <= hi), idx, S)

        def tok_loop(lo, hi_, a0, a1, a2, a3):
            def tok(i, accs):
                a0, a1, a2, a3 = accs
                li = i - s
                wgt = plsc.load_gather(w_v, [_splat(li)])
                po = plsc.load_gather(poff_v, [_splat(li)])[0]
                a0 = a0 + wgt * rows_v[li, pl.ds(po, LANES)]
                a1 = a1 + wgt * rows_v[li, pl.ds(po + LANES, LANES)]
                a2 = a2 + wgt * rows_v[li, pl.ds(po + 2 * LANES, LANES)]
                a3 = a3 + wgt * rows_v[li, pl.ds(po + 3 * LANES, LANES)]
                return (a0, a1, a2, a3)

            return lax.fori_loop(lo, hi_, tok, (a0, a1, a2, a3))

        def bag_body(k, st):
            p, a0, a1, a2, a3 = st
            nb = plsc.load_gather(offs_v, [_splat(k + 1)])[0]
            a0, a1, a2, a3 = tok_loop(p, nb, a0, a1, a2, a3)
            acc_v[k, pl.ds(0, LANES)] = a0
            acc_v[k, pl.ds(LANES, LANES)] = a1
            acc_v[k, pl.ds(2 * LANES, LANES)] = a2
            acc_v[k, pl.ds(3 * LANES, LANES)] = a3
            return (nb, zero16f, zero16f, zero16f, zero16f)

        p, a0, a1, a2, a3 = lax.fori_loop(cur, S - 1, bag_body,
                                          (p, a0, a1, a2, a3))
        cur = jnp.maximum(cur, S - 1)

        a0, a1, a2, a3 = tok_loop(p, hi, a0, a1, a2, a3)
        return (hi, cur, a0, a1, a2, a3)

    init = (t0, jnp.int32(0), zero16f, zero16f, zero16f, zero16f)
    p, cur, a0, a1, a2, a3 = lax.fori_loop(c_start, c_end, chunk_body, init)

    ci = jnp.minimum(cur, NBW)
    acc_v[ci, pl.ds(0, LANES)] = a0
    acc_v[ci, pl.ds(LANES, LANES)] = a1
    acc_v[ci, pl.ds(2 * LANES, LANES)] = a2
    acc_v[ci, pl.ds(3 * LANES, LANES)] = a3

    pltpu.sync_copy(acc_v.at[pl.ds(0, NBW)], out_hbm.at[pl.ds(b0, NBW)])


@functools.cache
def _build():
    mesh = plsc.VectorSubcoreMesh(core_axis_name="c", subcore_axis_name="s")
    return pl.kernel(
        _body,
        out_type=jax.ShapeDtypeStruct((N_BAGS, EMB_DIM), jnp.float32),
        mesh=mesh,
        scratch_types=[
            pltpu.VMEM((NBW,), jnp.int32),
            pltpu.VMEM((LANES,), jnp.int32),
            pltpu.VMEM((CHUNK,), jnp.int32),
            pltpu.VMEM((CHUNK,), jnp.int32),
            pltpu.VMEM((CHUNK,), jnp.int32),
            pltpu.VMEM((CHUNK,), jnp.float32),
            pltpu.VMEM((CHUNK, PAIR), jnp.float32),
            pltpu.VMEM((NBW + 1, EMB_DIM), jnp.float32),
            pltpu.SemaphoreType.DMA,
        ],
        compiler_params=pltpu.CompilerParams(needs_layout_passes=False),
        name="emb_bag_segment_sum",
    )


@jax.jit
def kernel(input_, offsets, emb_weights, emb_table):
    fn = _build()
    return fn(input_.astype(jnp.int32), offsets.astype(jnp.int32),
              emb_weights,
              emb_table.reshape(VOCAB // 2, PAIR))
